# Initial kernel scaffold; baseline (speedup 1.0000x reference)
import jax
import jax.numpy as jnp
from jax.experimental import pallas as pl


def kernel(h, mem2, mem3, mem4, ring_assign, Wih, Whh, bih, bhh, W1, b1, W2, b2):
    raise NotImplementedError("write your pallas kernel here")


# speedup vs baseline: 1.0528x; 1.0528x over previous
import functools

import jax
import jax.numpy as jnp
from jax.experimental import pallas as pl

U = 128
NR = 5000
_BLK = 1000


def _mm_bias_kernel(x_ref, wT_ref, b_ref, o_ref):
    o_ref[...] = (
        jnp.dot(x_ref[...], wT_ref[...], preferred_element_type=jnp.float32, precision=jax.lax.Precision.HIGHEST)
        + b_ref[0, :]
    )


def _mm_bias(x, wT, b, block=_BLK):
    n, kdim = x.shape
    odim = wT.shape[1]
    return pl.pallas_call(
        _mm_bias_kernel,
        grid=(n // block,),
        in_specs=[
            pl.BlockSpec((block, kdim), lambda i: (i, 0)),
            pl.BlockSpec((kdim, odim), lambda i: (0, 0)),
            pl.BlockSpec((1, odim), lambda i: (0, 0)),
        ],
        out_specs=pl.BlockSpec((block, odim), lambda i: (i, 0)),
        out_shape=jax.ShapeDtypeStruct((n, odim), jnp.float32),
    )(x, wT, b.reshape(1, odim))


def _gru_kernel(k, gi_ref, whhT_ref, bhh_ref, w1kT_ref, o_ref):
    bhh = bhh_ref[0, :]
    w1kT = w1kT_ref[...]
    h = None
    for t in range(k):
        gi = gi_ref[:, t * 3 * U:(t + 1) * 3 * U]
        if t == 0:
            ir, iz, inn = gi[:, :U], gi[:, U:2 * U], gi[:, 2 * U:]
            r = jax.nn.sigmoid(ir + bhh[:U])
            z = jax.nn.sigmoid(iz + bhh[U:2 * U])
            n = jnp.tanh(inn + r * bhh[2 * U:])
            h = (1.0 - z) * n
        else:
            gh = jnp.dot(h, whhT_ref[...], preferred_element_type=jnp.float32, precision=jax.lax.Precision.HIGHEST) + bhh
            r = jax.nn.sigmoid(gi[:, :U] + gh[:, :U])
            z = jax.nn.sigmoid(gi[:, U:2 * U] + gh[:, U:2 * U])
            n = jnp.tanh(gi[:, 2 * U:] + r * gh[:, 2 * U:])
            h = (1.0 - z) * n + z * h
        o_ref[:, t * U:(t + 1) * U] = jnp.dot(
            h, w1kT, preferred_element_type=jnp.float32,
            precision=jax.lax.Precision.HIGHEST,
        )


def _gru_project(k, gi_flat, whhT, bhh, w1kT, block=_BLK):
    nk = gi_flat.shape[0] // k
    gi = gi_flat.reshape(nk, k * 3 * U)
    return pl.pallas_call(
        functools.partial(_gru_kernel, k),
        grid=(nk // block,),
        in_specs=[
            pl.BlockSpec((block, k * 3 * U), lambda i: (i, 0)),
            pl.BlockSpec((U, 3 * U), lambda i: (0, 0)),
            pl.BlockSpec((1, 3 * U), lambda i: (0, 0)),
            pl.BlockSpec((U, U), lambda i: (0, 0)),
        ],
        out_specs=pl.BlockSpec((block, k * U), lambda i: (i, 0)),
        out_shape=jax.ShapeDtypeStruct((nk, k * U), jnp.float32),
    )(gi, whhT, bhh.reshape(1, 3 * U), w1kT)


def _final_kernel(acc_ref, w2T_ref, b2_ref, o_ref):
    o_ref[...] = (
        jnp.dot(jnp.tanh(acc_ref[...]), w2T_ref[...],
                preferred_element_type=jnp.float32, precision=jax.lax.Precision.HIGHEST)
        + b2_ref[0, :]
    )


def _final(acc, w2T, b2, block=_BLK):
    n = acc.shape[0]
    return pl.pallas_call(
        _final_kernel,
        grid=(n // block,),
        in_specs=[
            pl.BlockSpec((block, U), lambda i: (i, 0)),
            pl.BlockSpec((U, U), lambda i: (0, 0)),
            pl.BlockSpec((1, U), lambda i: (0, 0)),
        ],
        out_specs=pl.BlockSpec((block, U), lambda i: (i, 0)),
        out_shape=jax.ShapeDtypeStruct((n, U), jnp.float32),
    )(acc, w2T, b2.reshape(1, U))


def kernel(h, mem2, mem3, mem4, ring_assign, Wih, Whh, bih, bhh, W1, b1, W2, b2):
    wihT = Wih.T
    whhT = Whh.T
    w1T = W1.T
    w2T = W2.T

    acc = _mm_bias(h, w1T[:U, :], b1)
    ring_h = jax.ops.segment_sum(h, ring_assign, num_segments=NR)
    pring = _mm_bias(ring_h, w1T[U:2 * U, :], jnp.zeros((U,), jnp.float32))
    acc = acc + pring[ring_assign]

    for ki, (k, mem) in enumerate(((2, mem2), (3, mem3), (4, mem4))):
        idx = mem.reshape(-1)
        msgs = h[idx]
        gi = _mm_bias(msgs, wihT, bih)
        pk = _gru_project(k, gi, whhT, bhh, w1T[(2 + ki) * U:(3 + ki) * U, :])
        acc = acc.at[idx].add(pk.reshape(-1, U))

    return _final(acc, w2T, b2)


# speedup vs baseline: 1.1656x; 1.1071x over previous
import functools

import jax
import jax.numpy as jnp
from jax import lax
from jax.experimental import pallas as pl
from jax.experimental.pallas import tpu as pltpu
from jax.experimental.pallas import tpu_sc as plsc

U = 128
NR = 5000
_BLK = 1000

_PS = 50048
_PR = 5120
_CW = 32


def _sc_scatter_body(p2, idx2, p3, idx3, p4, idx4, hsrc, ridx, zeros_hbm,
                     s_out, r_out, sacc, racc, idx_v, data_v):
    c = lax.axis_index("c")
    s = lax.axis_index("s")
    sources = ((p2, idx2), (p3, idx3), (p4, idx4), (hsrc, ridx))
    for cc in range(2):
        col0 = (2 * c + cc) * _CW

        @pl.when(s == 0)
        def _():
            pltpu.sync_copy(zeros_hbm, sacc)

        @pl.when(s == 1)
        def _():
            pltpu.sync_copy(zeros_hbm.at[pl.ds(0, _PR)], racc)

        plsc.subcore_barrier()
        for src, idx2d in sources:
            nb = idx2d.shape[0]
            tail = src.shape[0] - (nb - 1) * 128
            acc = racc if src is hsrc else sacc
            nbf = nb - 1

            def body(i, carry, idx2d=idx2d, src=src, acc=acc, col0=col0):
                b = i * 16 + s
                pltpu.sync_copy(idx2d.at[b], idx_v.at[0])
                pltpu.sync_copy(
                    src.at[pl.ds(b * 128, 128), pl.ds(col0, _CW)], data_v)
                pltpu.sync_copy(data_v, acc.at[idx_v.at[0]], add=True)
                return carry

            lax.fori_loop(0, (nbf - s + 15) // 16, body, 0)

            @pl.when(s == (nbf % 16))
            def _(idx2d=idx2d, src=src, acc=acc, col0=col0, nbf=nbf,
                  tail=tail):
                pltpu.sync_copy(idx2d.at[nbf], idx_v.at[0])
                pltpu.sync_copy(
                    src.at[pl.ds(nbf * 128, tail), pl.ds(col0, _CW)],
                    data_v.at[pl.ds(0, tail)])
                pltpu.sync_copy(data_v, acc.at[idx_v.at[0]], add=True)

        plsc.subcore_barrier()

        @pl.when(s == 0)
        def _():
            pltpu.sync_copy(sacc, s_out.at[:, pl.ds(col0, _CW)])

        @pl.when(s == 1)
        def _():
            pltpu.sync_copy(racc, r_out.at[:, pl.ds(col0, _CW)])

        plsc.subcore_barrier()


def _sc_scatter(p2, i2, p3, i3, p4, i4, h, ra, zeros):
    return pl.kernel(
        _sc_scatter_body,
        out_type=(jax.ShapeDtypeStruct((_PS, 128), jnp.float32),
                  jax.ShapeDtypeStruct((_PR, 128), jnp.float32)),
        mesh=plsc.VectorSubcoreMesh(core_axis_name="c", subcore_axis_name="s"),
        compiler_params=pltpu.CompilerParams(use_tc_tiling_on_sc=False),
        scratch_types=[
            pltpu.VMEM_SHARED((_PS, _CW), jnp.float32),
            pltpu.VMEM_SHARED((_PR, _CW), jnp.float32),
            pltpu.VMEM((1, 128), jnp.int32),
            pltpu.VMEM((128, _CW), jnp.float32),
        ],
    )(p2, i2, p3, i3, p4, i4, h, ra, zeros)


def _pad2d(idx, sink):
    r = idx.shape[0]
    nb = -(-r // 128)
    return jnp.pad(idx, (0, nb * 128 - r), constant_values=sink).reshape(
        nb, 128)


def _mm_bias_kernel(x_ref, wT_ref, b_ref, o_ref):
    o_ref[...] = (
        jnp.dot(x_ref[...], wT_ref[...], preferred_element_type=jnp.float32, precision=jax.lax.Precision.HIGHEST)
        + b_ref[0, :]
    )


def _mm_bias(x, wT, b, block=_BLK):
    n, kdim = x.shape
    odim = wT.shape[1]
    return pl.pallas_call(
        _mm_bias_kernel,
        grid=(n // block,),
        in_specs=[
            pl.BlockSpec((block, kdim), lambda i: (i, 0)),
            pl.BlockSpec((kdim, odim), lambda i: (0, 0)),
            pl.BlockSpec((1, odim), lambda i: (0, 0)),
        ],
        out_specs=pl.BlockSpec((block, odim), lambda i: (i, 0)),
        out_shape=jax.ShapeDtypeStruct((n, odim), jnp.float32),
    )(x, wT, b.reshape(1, odim))


def _gru_kernel(k, gi_ref, whhT_ref, bhh_ref, w1kT_ref, o_ref):
    bhh = bhh_ref[0, :]
    w1kT = w1kT_ref[...]
    h = None
    for t in range(k):
        gi = gi_ref[:, t * 3 * U:(t + 1) * 3 * U]
        if t == 0:
            ir, iz, inn = gi[:, :U], gi[:, U:2 * U], gi[:, 2 * U:]
            r = jax.nn.sigmoid(ir + bhh[:U])
            z = jax.nn.sigmoid(iz + bhh[U:2 * U])
            n = jnp.tanh(inn + r * bhh[2 * U:])
            h = (1.0 - z) * n
        else:
            gh = jnp.dot(h, whhT_ref[...], preferred_element_type=jnp.float32, precision=jax.lax.Precision.HIGHEST) + bhh
            r = jax.nn.sigmoid(gi[:, :U] + gh[:, :U])
            z = jax.nn.sigmoid(gi[:, U:2 * U] + gh[:, U:2 * U])
            n = jnp.tanh(gi[:, 2 * U:] + r * gh[:, 2 * U:])
            h = (1.0 - z) * n + z * h
        o_ref[:, t * U:(t + 1) * U] = jnp.dot(
            h, w1kT, preferred_element_type=jnp.float32,
            precision=jax.lax.Precision.HIGHEST,
        )


def _gru_project(k, gi_flat, whhT, bhh, w1kT, block=_BLK):
    nk = gi_flat.shape[0] // k
    gi = gi_flat.reshape(nk, k * 3 * U)
    return pl.pallas_call(
        functools.partial(_gru_kernel, k),
        grid=(nk // block,),
        in_specs=[
            pl.BlockSpec((block, k * 3 * U), lambda i: (i, 0)),
            pl.BlockSpec((U, 3 * U), lambda i: (0, 0)),
            pl.BlockSpec((1, 3 * U), lambda i: (0, 0)),
            pl.BlockSpec((U, U), lambda i: (0, 0)),
        ],
        out_specs=pl.BlockSpec((block, k * U), lambda i: (i, 0)),
        out_shape=jax.ShapeDtypeStruct((nk, k * U), jnp.float32),
    )(gi, whhT, bhh.reshape(1, 3 * U), w1kT)


def _final_kernel(d_ref, g_ref, s_ref, w2T_ref, b2_ref, o_ref):
    pre = d_ref[...] + g_ref[...] + s_ref[...]
    o_ref[...] = (
        jnp.dot(jnp.tanh(pre), w2T_ref[...],
                preferred_element_type=jnp.float32,
                precision=jax.lax.Precision.HIGHEST)
        + b2_ref[0, :]
    )


def _final(d, g, s_pad, w2T, b2, block=_BLK):
    n = d.shape[0]
    return pl.pallas_call(
        _final_kernel,
        grid=(n // block,),
        in_specs=[
            pl.BlockSpec((block, U), lambda i: (i, 0)),
            pl.BlockSpec((block, U), lambda i: (i, 0)),
            pl.BlockSpec((block, U), lambda i: (i, 0)),
            pl.BlockSpec((U, U), lambda i: (0, 0)),
            pl.BlockSpec((1, U), lambda i: (0, 0)),
        ],
        out_specs=pl.BlockSpec((block, U), lambda i: (i, 0)),
        out_shape=jax.ShapeDtypeStruct((n, U), jnp.float32),
    )(d, g, s_pad, w2T, b2.reshape(1, U))


def kernel(h, mem2, mem3, mem4, ring_assign, Wih, Whh, bih, bhh, W1, b1, W2, b2):
    wihT = Wih.T
    whhT = Whh.T
    w1T = W1.T
    w2T = W2.T

    pks = []
    idxs = []
    for ki, (k, mem) in enumerate(((2, mem2), (3, mem3), (4, mem4))):
        idx = mem.reshape(-1)
        msgs = h[idx]
        gi = _mm_bias(msgs, wihT, bih)
        pk = _gru_project(k, gi, whhT, bhh, w1T[(2 + ki) * U:(3 + ki) * U, :])
        pks.append(pk.reshape(-1, U))
        idxs.append(_pad2d(idx, _PS - 48))

    zeros = jnp.zeros((_PS, _CW), jnp.float32)
    s_pad, r_pad = _sc_scatter(
        pks[0], idxs[0], pks[1], idxs[1], pks[2], idxs[2],
        h, _pad2d(ring_assign, NR), zeros)

    pring = _mm_bias(r_pad, w1T[U:2 * U, :], jnp.zeros((U,), jnp.float32),
                     block=640)
    d1 = _mm_bias(h, w1T[:U, :], b1)
    g = pring[ring_assign]
    return _final(d1, g, s_pad, w2T, b2)


# speedup vs baseline: 1.3237x; 1.1357x over previous
import functools

import jax
import jax.numpy as jnp
from jax import lax
from jax.experimental import pallas as pl
from jax.experimental.pallas import tpu as pltpu
from jax.experimental.pallas import tpu_sc as plsc

U = 128
NR = 5000
_BLK = 1000

_PS = 50048
_PR = 5120
_CW = 32


def _sc_scatter_body(p2, idx2, p3, idx3, p4, idx4, hsrc, ridx, zeros_hbm,
                     s_out, r_out, sacc, racc, idx_v, data_v):
    c = lax.axis_index("c")
    s = lax.axis_index("s")
    sources = ((p2, idx2), (p3, idx3), (p4, idx4), (hsrc, ridx))
    for cc in range(2):
        col0 = (2 * c + cc) * _CW

        @pl.when(s == 0)
        def _():
            pltpu.sync_copy(zeros_hbm, sacc)

        @pl.when(s == 1)
        def _():
            pltpu.sync_copy(zeros_hbm.at[pl.ds(0, _PR)], racc)

        plsc.subcore_barrier()
        for src, idx2d in sources:
            nb = idx2d.shape[0]
            tail = src.shape[0] - (nb - 1) * 128
            acc = racc if src is hsrc else sacc
            nbf = nb - 1

            def body(i, carry, idx2d=idx2d, src=src, acc=acc, col0=col0):
                b = i * 16 + s
                pltpu.sync_copy(idx2d.at[b], idx_v.at[0])
                pltpu.sync_copy(
                    src.at[pl.ds(b * 128, 128), pl.ds(col0, _CW)], data_v)
                pltpu.sync_copy(data_v, acc.at[idx_v.at[0]], add=True)
                return carry

            lax.fori_loop(0, (nbf - s + 15) // 16, body, 0)

            @pl.when(s == (nbf % 16))
            def _(idx2d=idx2d, src=src, acc=acc, col0=col0, nbf=nbf,
                  tail=tail):
                pltpu.sync_copy(idx2d.at[nbf], idx_v.at[0])
                pltpu.sync_copy(
                    src.at[pl.ds(nbf * 128, tail), pl.ds(col0, _CW)],
                    data_v.at[pl.ds(0, tail)])
                pltpu.sync_copy(data_v, acc.at[idx_v.at[0]], add=True)

        plsc.subcore_barrier()

        @pl.when(s == 0)
        def _():
            pltpu.sync_copy(sacc, s_out.at[:, pl.ds(col0, _CW)])

        @pl.when(s == 1)
        def _():
            pltpu.sync_copy(racc, r_out.at[:, pl.ds(col0, _CW)])

        plsc.subcore_barrier()


def _sc_scatter(p2, i2, p3, i3, p4, i4, h, ra, zeros):
    return pl.kernel(
        _sc_scatter_body,
        out_type=(jax.ShapeDtypeStruct((_PS, 128), jnp.float32),
                  jax.ShapeDtypeStruct((_PR, 128), jnp.float32)),
        mesh=plsc.VectorSubcoreMesh(core_axis_name="c", subcore_axis_name="s"),
        compiler_params=pltpu.CompilerParams(use_tc_tiling_on_sc=False),
        scratch_types=[
            pltpu.VMEM_SHARED((_PS, _CW), jnp.float32),
            pltpu.VMEM_SHARED((_PR, _CW), jnp.float32),
            pltpu.VMEM((1, 128), jnp.int32),
            pltpu.VMEM((128, _CW), jnp.float32),
        ],
    )(p2, i2, p3, i3, p4, i4, h, ra, zeros)


def _pad2d(idx, sink):
    r = idx.shape[0]
    nb = -(-r // 128)
    return jnp.pad(idx, (0, nb * 128 - r), constant_values=sink).reshape(
        nb, 128)


def _mm_bias_kernel(x_ref, wT_ref, b_ref, o_ref):
    o_ref[...] = (
        jnp.dot(x_ref[...], wT_ref[...], preferred_element_type=jnp.float32, precision=jax.lax.Precision.HIGHEST)
        + b_ref[0, :]
    )


def _mm_bias(x, wT, b, block=_BLK):
    n, kdim = x.shape
    odim = wT.shape[1]
    return pl.pallas_call(
        _mm_bias_kernel,
        grid=(n // block,),
        in_specs=[
            pl.BlockSpec((block, kdim), lambda i: (i, 0)),
            pl.BlockSpec((kdim, odim), lambda i: (0, 0)),
            pl.BlockSpec((1, odim), lambda i: (0, 0)),
        ],
        out_specs=pl.BlockSpec((block, odim), lambda i: (i, 0)),
        out_shape=jax.ShapeDtypeStruct((n, odim), jnp.float32),
    )(x, wT, b.reshape(1, odim))


def _gru_kernel(k, m_ref, wihT_ref, bih_ref, whhT_ref, bhh_ref, w1kT_ref,
                o_ref):
    bhh = bhh_ref[0, :]
    w1kT = w1kT_ref[...]
    wihT = wihT_ref[...]
    h = None
    for t in range(k):
        gi = jnp.dot(m_ref[:, t * U:(t + 1) * U], wihT,
                     preferred_element_type=jnp.float32,
                     precision=jax.lax.Precision.HIGHEST) + bih_ref[0, :]
        if t == 0:
            ir, iz, inn = gi[:, :U], gi[:, U:2 * U], gi[:, 2 * U:]
            r = jax.nn.sigmoid(ir + bhh[:U])
            z = jax.nn.sigmoid(iz + bhh[U:2 * U])
            n = jnp.tanh(inn + r * bhh[2 * U:])
            h = (1.0 - z) * n
        else:
            gh = jnp.dot(h, whhT_ref[...], preferred_element_type=jnp.float32, precision=jax.lax.Precision.HIGHEST) + bhh
            r = jax.nn.sigmoid(gi[:, :U] + gh[:, :U])
            z = jax.nn.sigmoid(gi[:, U:2 * U] + gh[:, U:2 * U])
            n = jnp.tanh(gi[:, 2 * U:] + r * gh[:, 2 * U:])
            h = (1.0 - z) * n + z * h
        o_ref[:, t * U:(t + 1) * U] = jnp.dot(
            h, w1kT, preferred_element_type=jnp.float32,
            precision=jax.lax.Precision.HIGHEST,
        )


def _gru_project(k, msgs_flat, wihT, bih, whhT, bhh, w1kT, block=_BLK):
    nk = msgs_flat.shape[0] // k
    msgs = msgs_flat.reshape(nk, k * U)
    return pl.pallas_call(
        functools.partial(_gru_kernel, k),
        grid=(nk // block,),
        in_specs=[
            pl.BlockSpec((block, k * U), lambda i: (i, 0)),
            pl.BlockSpec((U, 3 * U), lambda i: (0, 0)),
            pl.BlockSpec((1, 3 * U), lambda i: (0, 0)),
            pl.BlockSpec((U, 3 * U), lambda i: (0, 0)),
            pl.BlockSpec((1, 3 * U), lambda i: (0, 0)),
            pl.BlockSpec((U, U), lambda i: (0, 0)),
        ],
        out_specs=pl.BlockSpec((block, k * U), lambda i: (i, 0)),
        out_shape=jax.ShapeDtypeStruct((nk, k * U), jnp.float32),
    )(msgs, wihT, bih.reshape(1, 3 * U), whhT, bhh.reshape(1, 3 * U), w1kT)


def _final_kernel(d_ref, g_ref, s_ref, w2T_ref, b2_ref, o_ref):
    pre = d_ref[...] + g_ref[...] + s_ref[...]
    o_ref[...] = (
        jnp.dot(jnp.tanh(pre), w2T_ref[...],
                preferred_element_type=jnp.float32,
                precision=jax.lax.Precision.HIGHEST)
        + b2_ref[0, :]
    )


def _final(d, g, s_pad, w2T, b2, block=_BLK):
    n = d.shape[0]
    return pl.pallas_call(
        _final_kernel,
        grid=(n // block,),
        in_specs=[
            pl.BlockSpec((block, U), lambda i: (i, 0)),
            pl.BlockSpec((block, U), lambda i: (i, 0)),
            pl.BlockSpec((block, U), lambda i: (i, 0)),
            pl.BlockSpec((U, U), lambda i: (0, 0)),
            pl.BlockSpec((1, U), lambda i: (0, 0)),
        ],
        out_specs=pl.BlockSpec((block, U), lambda i: (i, 0)),
        out_shape=jax.ShapeDtypeStruct((n, U), jnp.float32),
    )(d, g, s_pad, w2T, b2.reshape(1, U))


def kernel(h, mem2, mem3, mem4, ring_assign, Wih, Whh, bih, bhh, W1, b1, W2, b2):
    wihT = Wih.T
    whhT = Whh.T
    w1T = W1.T
    w2T = W2.T

    pks = []
    idxs = []
    for ki, (k, mem) in enumerate(((2, mem2), (3, mem3), (4, mem4))):
        idx = mem.reshape(-1)
        msgs = h[idx]
        pk = _gru_project(k, msgs, wihT, bih, whhT, bhh,
                          w1T[(2 + ki) * U:(3 + ki) * U, :])
        pks.append(pk.reshape(-1, U))
        idxs.append(_pad2d(idx, _PS - 48))

    zeros = jnp.zeros((_PS, _CW), jnp.float32)
    s_pad, r_pad = _sc_scatter(
        pks[0], idxs[0], pks[1], idxs[1], pks[2], idxs[2],
        h, _pad2d(ring_assign, NR), zeros)

    pring = _mm_bias(r_pad, w1T[U:2 * U, :], jnp.zeros((U,), jnp.float32),
                     block=640)
    d1 = _mm_bias(h, w1T[:U, :], b1)
    g = pring[ring_assign]
    return _final(d1, g, s_pad, w2T, b2)


# speedup vs baseline: 1.6948x; 1.2803x over previous
import functools

import jax
import jax.numpy as jnp
from jax import lax
from jax.experimental import pallas as pl
from jax.experimental.pallas import tpu as pltpu
from jax.experimental.pallas import tpu_sc as plsc

U = 128
NR = 5000
_BLK = 1000

_PS = 50048
_PR = 5120
_CW = 32


def _sc_scatter_body(p2, idx2, p3, idx3, p4, idx4, hsrc, ridx, zeros_hbm,
                     s_out, r_out, sacc, racc, idx_v, data_v):
    c = lax.axis_index("c")
    s = lax.axis_index("s")
    sources = ((p2, idx2), (p3, idx3), (p4, idx4), (hsrc, ridx))
    for cc in range(2):
        col0 = (2 * c + cc) * _CW

        @pl.when(s == 0)
        def _():
            pltpu.sync_copy(zeros_hbm, sacc)

        @pl.when(s == 1)
        def _():
            pltpu.sync_copy(zeros_hbm.at[pl.ds(0, _PR)], racc)

        plsc.subcore_barrier()
        for src, idx2d in sources:
            nb = idx2d.shape[0]
            tail = src.shape[0] - (nb - 1) * 128
            acc = racc if src is hsrc else sacc
            nbf = nb - 1

            def body(i, carry, idx2d=idx2d, src=src, acc=acc, col0=col0):
                b = i * 16 + s
                pltpu.sync_copy(idx2d.at[b], idx_v.at[0])
                pltpu.sync_copy(
                    src.at[pl.ds(b * 128, 128), pl.ds(col0, _CW)], data_v)
                pltpu.sync_copy(data_v, acc.at[idx_v.at[0]], add=True)
                return carry

            lax.fori_loop(0, (nbf - s + 15) // 16, body, 0)

            @pl.when(s == (nbf % 16))
            def _(idx2d=idx2d, src=src, acc=acc, col0=col0, nbf=nbf,
                  tail=tail):
                pltpu.sync_copy(idx2d.at[nbf], idx_v.at[0])
                pltpu.sync_copy(
                    src.at[pl.ds(nbf * 128, tail), pl.ds(col0, _CW)],
                    data_v.at[pl.ds(0, tail)])
                pltpu.sync_copy(data_v, acc.at[idx_v.at[0]], add=True)

        plsc.subcore_barrier()

        @pl.when(s == 0)
        def _():
            pltpu.sync_copy(sacc, s_out.at[:, pl.ds(col0, _CW)])

        @pl.when(s == 1)
        def _():
            pltpu.sync_copy(racc, r_out.at[:, pl.ds(col0, _CW)])

        plsc.subcore_barrier()


def _sc_scatter(p2, i2, p3, i3, p4, i4, h, ra, zeros):
    return pl.kernel(
        _sc_scatter_body,
        out_type=(jax.ShapeDtypeStruct((_PS, 128), jnp.float32),
                  jax.ShapeDtypeStruct((_PR, 128), jnp.float32)),
        mesh=plsc.VectorSubcoreMesh(core_axis_name="c", subcore_axis_name="s"),
        compiler_params=pltpu.CompilerParams(use_tc_tiling_on_sc=False),
        scratch_types=[
            pltpu.VMEM_SHARED((_PS, _CW), jnp.float32),
            pltpu.VMEM_SHARED((_PR, _CW), jnp.float32),
            pltpu.VMEM((1, 128), jnp.int32),
            pltpu.VMEM((128, _CW), jnp.float32),
        ],
    )(p2, i2, p3, i3, p4, i4, h, ra, zeros)


def _pad2d(idx, sink):
    r = idx.shape[0]
    nb = -(-r // 128)
    return jnp.pad(idx, (0, nb * 128 - r), constant_values=sink).reshape(
        nb, 128)


def _mm_bias_kernel(x_ref, wT_ref, b_ref, o_ref):
    o_ref[...] = (
        jnp.dot(x_ref[...], wT_ref[...], preferred_element_type=jnp.float32, precision=jax.lax.Precision.HIGHEST)
        + b_ref[0, :]
    )


def _mm_bias(x, wT, b, block=_BLK):
    n, kdim = x.shape
    odim = wT.shape[1]
    return pl.pallas_call(
        _mm_bias_kernel,
        grid=(n // block,),
        in_specs=[
            pl.BlockSpec((block, kdim), lambda i: (i, 0)),
            pl.BlockSpec((kdim, odim), lambda i: (0, 0)),
            pl.BlockSpec((1, odim), lambda i: (0, 0)),
        ],
        out_specs=pl.BlockSpec((block, odim), lambda i: (i, 0)),
        out_shape=jax.ShapeDtypeStruct((n, odim), jnp.float32),
    )(x, wT, b.reshape(1, odim))


def _split_bf16(w):
    hi = w.astype(jnp.bfloat16)
    lo = (w - hi.astype(jnp.float32)).astype(jnp.bfloat16)
    return hi, lo


def _dot3(a, w_hi, w_lo):
    a_hi = a.astype(jnp.bfloat16)
    a_lo = (a - a_hi.astype(jnp.float32)).astype(jnp.bfloat16)
    return (jnp.dot(a_hi, w_hi, preferred_element_type=jnp.float32)
            + (jnp.dot(a_hi, w_lo, preferred_element_type=jnp.float32)
               + jnp.dot(a_lo, w_hi, preferred_element_type=jnp.float32)))


def _gru_kernel(k, m_ref, wih_h, wih_l, bih_ref, whh_h, whh_l, bhh_ref,
                w1k_h, w1k_l, o_ref):
    bhh = bhh_ref[0, :]
    h = None
    for t in range(k):
        gi = _dot3(m_ref[:, t * U:(t + 1) * U], wih_h[...], wih_l[...]) \
            + bih_ref[0, :]
        if t == 0:
            ir, iz, inn = gi[:, :U], gi[:, U:2 * U], gi[:, 2 * U:]
            r = jax.nn.sigmoid(ir + bhh[:U])
            z = jax.nn.sigmoid(iz + bhh[U:2 * U])
            n = jnp.tanh(inn + r * bhh[2 * U:])
            h = (1.0 - z) * n
        else:
            gh = _dot3(h, whh_h[...], whh_l[...]) + bhh
            r = jax.nn.sigmoid(gi[:, :U] + gh[:, :U])
            z = jax.nn.sigmoid(gi[:, U:2 * U] + gh[:, U:2 * U])
            n = jnp.tanh(gi[:, 2 * U:] + r * gh[:, 2 * U:])
            h = (1.0 - z) * n + z * h
        o_ref[:, t * U:(t + 1) * U] = _dot3(h, w1k_h[...], w1k_l[...])


def _gru_project(k, msgs_flat, wihT, bih, whhT, bhh, w1kT, block=_BLK):
    nk = msgs_flat.shape[0] // k
    msgs = msgs_flat.reshape(nk, k * U)
    wih_h, wih_l = _split_bf16(wihT)
    whh_h, whh_l = _split_bf16(whhT)
    w1k_h, w1k_l = _split_bf16(w1kT)
    wspec = pl.BlockSpec((U, 3 * U), lambda i: (0, 0))
    bspec = pl.BlockSpec((1, 3 * U), lambda i: (0, 0))
    pspec = pl.BlockSpec((U, U), lambda i: (0, 0))
    return pl.pallas_call(
        functools.partial(_gru_kernel, k),
        grid=(nk // block,),
        in_specs=[
            pl.BlockSpec((block, k * U), lambda i: (i, 0)),
            wspec, wspec, bspec, wspec, wspec, bspec, pspec, pspec,
        ],
        out_specs=pl.BlockSpec((block, k * U), lambda i: (i, 0)),
        out_shape=jax.ShapeDtypeStruct((nk, k * U), jnp.float32),
    )(msgs, wih_h, wih_l, bih.reshape(1, 3 * U), whh_h, whh_l,
      bhh.reshape(1, 3 * U), w1k_h, w1k_l)


def _final_kernel(d_ref, g_ref, s_ref, w2T_ref, b2_ref, o_ref):
    pre = d_ref[...] + g_ref[...] + s_ref[...]
    o_ref[...] = (
        jnp.dot(jnp.tanh(pre), w2T_ref[...],
                preferred_element_type=jnp.float32,
                precision=jax.lax.Precision.HIGHEST)
        + b2_ref[0, :]
    )


def _final(d, g, s_pad, w2T, b2, block=_BLK):
    n = d.shape[0]
    return pl.pallas_call(
        _final_kernel,
        grid=(n // block,),
        in_specs=[
            pl.BlockSpec((block, U), lambda i: (i, 0)),
            pl.BlockSpec((block, U), lambda i: (i, 0)),
            pl.BlockSpec((block, U), lambda i: (i, 0)),
            pl.BlockSpec((U, U), lambda i: (0, 0)),
            pl.BlockSpec((1, U), lambda i: (0, 0)),
        ],
        out_specs=pl.BlockSpec((block, U), lambda i: (i, 0)),
        out_shape=jax.ShapeDtypeStruct((n, U), jnp.float32),
    )(d, g, s_pad, w2T, b2.reshape(1, U))


def kernel(h, mem2, mem3, mem4, ring_assign, Wih, Whh, bih, bhh, W1, b1, W2, b2):
    wihT = Wih.T
    whhT = Whh.T
    w1T = W1.T
    w2T = W2.T

    pks = []
    idxs = []
    for ki, (k, mem) in enumerate(((2, mem2), (3, mem3), (4, mem4))):
        idx = mem.reshape(-1)
        msgs = h[idx]
        pk = _gru_project(k, msgs, wihT, bih, whhT, bhh,
                          w1T[(2 + ki) * U:(3 + ki) * U, :])
        pks.append(pk.reshape(-1, U))
        idxs.append(_pad2d(idx, _PS - 48))

    zeros = jnp.zeros((_PS, _CW), jnp.float32)
    s_pad, r_pad = _sc_scatter(
        pks[0], idxs[0], pks[1], idxs[1], pks[2], idxs[2],
        h, _pad2d(ring_assign, NR), zeros)

    pring = _mm_bias(r_pad, w1T[U:2 * U, :], jnp.zeros((U,), jnp.float32),
                     block=640)
    d1 = _mm_bias(h, w1T[:U, :], b1)
    g = pring[ring_assign]
    return _final(d1, g, s_pad, w2T, b2)


# speedup vs baseline: 1.9208x; 1.1334x over previous
import functools

import jax
import jax.numpy as jnp
from jax import lax
from jax.experimental import pallas as pl
from jax.experimental.pallas import tpu as pltpu
from jax.experimental.pallas import tpu_sc as plsc

U = 128
NR = 5000
_BLK = 1000

_PS = 50048
_PR = 5120
_CW = 32


_NBP = 196
_NBR = 390


def _sc_scatter_body(*refs):
    ps = refs[0:9]
    idxs = refs[9:18]
    hsrc, ridx, zeros_hbm, s_out, r_out = refs[18:23]
    sacc, racc, idx_v, data_v = refs[23:27]
    c = lax.axis_index("c")
    s = lax.axis_index("s")
    for cc in range(2):
        col0 = (2 * c + cc) * _CW

        @pl.when(s == 0)
        def _():
            pltpu.sync_copy(zeros_hbm, sacc)

        @pl.when(s == 1)
        def _():
            pltpu.sync_copy(zeros_hbm.at[pl.ds(0, _PR)], racc)

        plsc.subcore_barrier()
        per, rem = _NBP // 16, _NBP % 16
        start_w = s * per + jnp.minimum(s, rem)
        nb_w = per + (s < rem).astype(jnp.int32)
        for src, idx2d in zip(ps, idxs):

            def body(i, carry, src=src, col0=col0, start_w=start_w):
                b = start_w + i
                pltpu.sync_copy(
                    src.at[pl.ds(b * 128, 128), pl.ds(col0, _CW)], data_v)
                pltpu.sync_copy(data_v, sacc.at[idx_v.at[i]], add=True)
                return carry

            pltpu.sync_copy(idx2d.at[pl.ds(start_w, per + 1)],
                            idx_v.at[pl.ds(0, per + 1)])
            lax.fori_loop(0, nb_w, body, 0)

        perr, remr = _NBR // 16, _NBR % 16
        start_r = s * perr + jnp.minimum(s, remr)
        nbr_w = perr + (s < remr).astype(jnp.int32)
        pltpu.sync_copy(ridx.at[pl.ds(start_r, perr + 1)],
                        idx_v.at[pl.ds(0, perr + 1)])

        def rbody(i, carry, col0=col0, start_r=start_r):
            b = start_r + i
            pltpu.sync_copy(
                hsrc.at[pl.ds(b * 128, 128), pl.ds(col0, _CW)], data_v)
            pltpu.sync_copy(data_v, racc.at[idx_v.at[i]], add=True)
            return carry

        lax.fori_loop(0, nbr_w, rbody, 0)

        @pl.when(s == 15)
        def _(col0=col0):
            pltpu.sync_copy(ridx.at[pl.ds(_NBR, 1)], idx_v.at[pl.ds(25, 1)])
            pltpu.sync_copy(
                hsrc.at[pl.ds(_NBR * 128, 80), pl.ds(col0, _CW)],
                data_v.at[pl.ds(0, 80)])
            pltpu.sync_copy(data_v, racc.at[idx_v.at[25]], add=True)

        plsc.subcore_barrier()

        @pl.when(s == 0)
        def _():
            pltpu.sync_copy(sacc, s_out.at[:, pl.ds(col0, _CW)])

        @pl.when(s == 1)
        def _():
            pltpu.sync_copy(racc, r_out.at[:, pl.ds(col0, _CW)])

        plsc.subcore_barrier()


def _sc_scatter(ps, idxs, h, ra, zeros):
    return pl.kernel(
        _sc_scatter_body,
        out_type=(jax.ShapeDtypeStruct((_PS, 128), jnp.float32),
                  jax.ShapeDtypeStruct((_PR, 128), jnp.float32)),
        mesh=plsc.VectorSubcoreMesh(core_axis_name="c", subcore_axis_name="s"),
        compiler_params=pltpu.CompilerParams(use_tc_tiling_on_sc=False),
        scratch_types=[
            pltpu.VMEM_SHARED((_PS, _CW), jnp.float32),
            pltpu.VMEM_SHARED((_PR, _CW), jnp.float32),
            pltpu.VMEM((26, 128), jnp.int32),
            pltpu.VMEM((128, _CW), jnp.float32),
        ],
    )(*ps, *idxs, h, ra, zeros)


def _pad2d(idx, nb, sink):
    return jnp.pad(idx, (0, (nb + 1) * 128 - idx.shape[0]),
                   constant_values=sink).reshape(nb + 1, 128)


def _mm_bias_kernel(x_ref, wT_ref, b_ref, o_ref):
    o_ref[...] = (
        jnp.dot(x_ref[...], wT_ref[...], preferred_element_type=jnp.float32, precision=jax.lax.Precision.HIGHEST)
        + b_ref[0, :]
    )


def _mm_bias(x, wT, b, block=_BLK):
    n, kdim = x.shape
    odim = wT.shape[1]
    return pl.pallas_call(
        _mm_bias_kernel,
        grid=(n // block,),
        in_specs=[
            pl.BlockSpec((block, kdim), lambda i: (i, 0)),
            pl.BlockSpec((kdim, odim), lambda i: (0, 0)),
            pl.BlockSpec((1, odim), lambda i: (0, 0)),
        ],
        out_specs=pl.BlockSpec((block, odim), lambda i: (i, 0)),
        out_shape=jax.ShapeDtypeStruct((n, odim), jnp.float32),
    )(x, wT, b.reshape(1, odim))


def _split_bf16(w):
    hi = w.astype(jnp.bfloat16)
    lo = (w - hi.astype(jnp.float32)).astype(jnp.bfloat16)
    return hi, lo


def _dot3(a, w_hi, w_lo):
    a_hi = a.astype(jnp.bfloat16)
    a_lo = (a - a_hi.astype(jnp.float32)).astype(jnp.bfloat16)
    return (jnp.dot(a_hi, w_hi, preferred_element_type=jnp.float32)
            + (jnp.dot(a_hi, w_lo, preferred_element_type=jnp.float32)
               + jnp.dot(a_lo, w_hi, preferred_element_type=jnp.float32)))


def _gru_kernel(k, m_ref, wih_h, wih_l, bih_ref, whh_h, whh_l, bhh_ref,
                w1k_h, w1k_l, *o_refs):
    bhh = bhh_ref[0, :]
    h = None
    for t in range(k):
        gi = _dot3(m_ref[:, t * U:(t + 1) * U], wih_h[...], wih_l[...]) \
            + bih_ref[0, :]
        if t == 0:
            ir, iz, inn = gi[:, :U], gi[:, U:2 * U], gi[:, 2 * U:]
            r = jax.nn.sigmoid(ir + bhh[:U])
            z = jax.nn.sigmoid(iz + bhh[U:2 * U])
            n = jnp.tanh(inn + r * bhh[2 * U:])
            h = (1.0 - z) * n
        else:
            gh = _dot3(h, whh_h[...], whh_l[...]) + bhh
            r = jax.nn.sigmoid(gi[:, :U] + gh[:, :U])
            z = jax.nn.sigmoid(gi[:, U:2 * U] + gh[:, U:2 * U])
            n = jnp.tanh(gi[:, 2 * U:] + r * gh[:, 2 * U:])
            h = (1.0 - z) * n + z * h
        o_refs[t][...] = _dot3(h, w1k_h[...], w1k_l[...])


def _gru_project(k, msgs_flat, wihT, bih, whhT, bhh, w1kT, block=_BLK):
    nk = msgs_flat.shape[0] // k
    msgs = msgs_flat.reshape(nk, k * U)
    wih_h, wih_l = _split_bf16(wihT)
    whh_h, whh_l = _split_bf16(whhT)
    w1k_h, w1k_l = _split_bf16(w1kT)
    wspec = pl.BlockSpec((U, 3 * U), lambda i: (0, 0))
    bspec = pl.BlockSpec((1, 3 * U), lambda i: (0, 0))
    pspec = pl.BlockSpec((U, U), lambda i: (0, 0))
    ospec = pl.BlockSpec((block, U), lambda i: (i, 0))
    return pl.pallas_call(
        functools.partial(_gru_kernel, k),
        grid=(nk // block,),
        in_specs=[
            pl.BlockSpec((block, k * U), lambda i: (i, 0)),
            wspec, wspec, bspec, wspec, wspec, bspec, pspec, pspec,
        ],
        out_specs=[ospec] * k,
        out_shape=[jax.ShapeDtypeStruct((_NBP * 128, U), jnp.float32)] * k,
    )(msgs, wih_h, wih_l, bih.reshape(1, 3 * U), whh_h, whh_l,
      bhh.reshape(1, 3 * U), w1k_h, w1k_l)


def _final_kernel(d_ref, g_ref, s_ref, w2T_ref, b2_ref, o_ref):
    pre = d_ref[...] + g_ref[...] + s_ref[...]
    o_ref[...] = (
        jnp.dot(jnp.tanh(pre), w2T_ref[...],
                preferred_element_type=jnp.float32,
                precision=jax.lax.Precision.HIGHEST)
        + b2_ref[0, :]
    )


def _final(d, g, s_pad, w2T, b2, block=_BLK):
    n = d.shape[0]
    return pl.pallas_call(
        _final_kernel,
        grid=(n // block,),
        in_specs=[
            pl.BlockSpec((block, U), lambda i: (i, 0)),
            pl.BlockSpec((block, U), lambda i: (i, 0)),
            pl.BlockSpec((block, U), lambda i: (i, 0)),
            pl.BlockSpec((U, U), lambda i: (0, 0)),
            pl.BlockSpec((1, U), lambda i: (0, 0)),
        ],
        out_specs=pl.BlockSpec((block, U), lambda i: (i, 0)),
        out_shape=jax.ShapeDtypeStruct((n, U), jnp.float32),
    )(d, g, s_pad, w2T, b2.reshape(1, U))


def kernel(h, mem2, mem3, mem4, ring_assign, Wih, Whh, bih, bhh, W1, b1, W2, b2):
    wihT = Wih.T
    whhT = Whh.T
    w1T = W1.T
    w2T = W2.T

    pks = []
    idxs = []
    for ki, (k, mem) in enumerate(((2, mem2), (3, mem3), (4, mem4))):
        msgs = h[mem.reshape(-1)]
        pk_ts = _gru_project(k, msgs, wihT, bih, whhT, bhh,
                             w1T[(2 + ki) * U:(3 + ki) * U, :])
        pks.extend(pk_ts)
        idxs.extend(_pad2d(mem[:, t], _NBP, _PS - 48) for t in range(k))

    zeros = jnp.zeros((_PS, _CW), jnp.float32)
    s_pad, r_pad = _sc_scatter(
        pks, idxs, h, _pad2d(ring_assign, _NBR + 1, NR), zeros)

    pring = _mm_bias(r_pad, w1T[U:2 * U, :], jnp.zeros((U,), jnp.float32),
                     block=640)
    d1 = _mm_bias(h, w1T[:U, :], b1)
    g = pring[ring_assign]
    return _final(d1, g, s_pad, w2T, b2)


# speedup vs baseline: 2.5895x; 1.3481x over previous
import functools

import jax
import jax.numpy as jnp
from jax import lax
from jax.experimental import pallas as pl
from jax.experimental.pallas import tpu as pltpu
from jax.experimental.pallas import tpu_sc as plsc

U = 128
NR = 5000
_BLK = 1000

_PS = 50048
_PR = 5120
_CW = 32


_NBP = 196
_NBR = 390


def _sc_gather_body(k, *refs):
    h = refs[0]
    idxs = refs[1:1 + k]
    outs = refs[1 + k:1 + 2 * k]
    idx_v, data_v, sem = refs[1 + 2 * k:]
    w = lax.axis_index("s") * 2 + lax.axis_index("c")
    per, rem = _NBP // 32, _NBP % 32
    start_w = w * per + jnp.minimum(w, rem)
    nb_w = per + (w < rem).astype(jnp.int32)
    for idx2d, out in zip(idxs, outs):
        pltpu.sync_copy(idx2d.at[pl.ds(start_w, per + 1)],
                        idx_v.at[pl.ds(0, per + 1)])

        def body(i, carry, out=out):
            b = start_w + i
            pltpu.async_copy(h.at[idx_v.at[i]], data_v, sem).wait()
            pltpu.sync_copy(data_v, out.at[pl.ds(b * 128, 128)])
            return carry

        lax.fori_loop(0, nb_w, body, 0)


def _sc_gather(k, h, idxs):
    return pl.kernel(
        functools.partial(_sc_gather_body, k),
        out_type=tuple(jax.ShapeDtypeStruct((_NBP * 128, 128), jnp.float32)
                       for _ in range(k)),
        mesh=plsc.VectorSubcoreMesh(core_axis_name="c", subcore_axis_name="s"),
        compiler_params=pltpu.CompilerParams(use_tc_tiling_on_sc=False),
        scratch_types=[
            pltpu.VMEM((7, 128), jnp.int32),
            pltpu.VMEM((128, 128), jnp.float32),
            pltpu.SemaphoreType.DMA,
        ],
    )(h, *idxs)


def _sc_scatter_body(*refs):
    ps = refs[0:9]
    idxs = refs[9:18]
    hsrc, ridx, zeros_hbm, s_out, r_out = refs[18:23]
    sacc, racc, idx_v, data_v = refs[23:27]
    c = lax.axis_index("c")
    s = lax.axis_index("s")
    for cc in range(2):
        col0 = (2 * c + cc) * _CW

        @pl.when(s == 0)
        def _():
            pltpu.sync_copy(zeros_hbm, sacc)

        @pl.when(s == 1)
        def _():
            pltpu.sync_copy(zeros_hbm.at[pl.ds(0, _PR)], racc)

        plsc.subcore_barrier()
        per, rem = _NBP // 16, _NBP % 16
        start_w = s * per + jnp.minimum(s, rem)
        nb_w = per + (s < rem).astype(jnp.int32)
        for src, idx2d in zip(ps, idxs):

            def body(i, carry, src=src, col0=col0, start_w=start_w):
                b = start_w + i
                pltpu.sync_copy(
                    src.at[pl.ds(b * 128, 128), pl.ds(col0, _CW)], data_v)
                pltpu.sync_copy(data_v, sacc.at[idx_v.at[i]], add=True)
                return carry

            pltpu.sync_copy(idx2d.at[pl.ds(start_w, per + 1)],
                            idx_v.at[pl.ds(0, per + 1)])
            lax.fori_loop(0, nb_w, body, 0)

        perr, remr = _NBR // 16, _NBR % 16
        start_r = s * perr + jnp.minimum(s, remr)
        nbr_w = perr + (s < remr).astype(jnp.int32)
        pltpu.sync_copy(ridx.at[pl.ds(start_r, perr + 1)],
                        idx_v.at[pl.ds(0, perr + 1)])

        def rbody(i, carry, col0=col0, start_r=start_r):
            b = start_r + i
            pltpu.sync_copy(
                hsrc.at[pl.ds(b * 128, 128), pl.ds(col0, _CW)], data_v)
            pltpu.sync_copy(data_v, racc.at[idx_v.at[i]], add=True)
            return carry

        lax.fori_loop(0, nbr_w, rbody, 0)

        @pl.when(s == 15)
        def _(col0=col0):
            pltpu.sync_copy(ridx.at[pl.ds(_NBR, 1)], idx_v.at[pl.ds(25, 1)])
            pltpu.sync_copy(
                hsrc.at[pl.ds(_NBR * 128, 80), pl.ds(col0, _CW)],
                data_v.at[pl.ds(0, 80)])
            pltpu.sync_copy(data_v, racc.at[idx_v.at[25]], add=True)

        plsc.subcore_barrier()

        @pl.when(s == 0)
        def _():
            pltpu.sync_copy(sacc, s_out.at[:, pl.ds(col0, _CW)])

        @pl.when(s == 1)
        def _():
            pltpu.sync_copy(racc, r_out.at[:, pl.ds(col0, _CW)])

        plsc.subcore_barrier()


def _sc_scatter(ps, idxs, h, ra, zeros):
    return pl.kernel(
        _sc_scatter_body,
        out_type=(jax.ShapeDtypeStruct((_PS, 128), jnp.float32),
                  jax.ShapeDtypeStruct((_PR, 128), jnp.float32)),
        mesh=plsc.VectorSubcoreMesh(core_axis_name="c", subcore_axis_name="s"),
        compiler_params=pltpu.CompilerParams(use_tc_tiling_on_sc=False),
        scratch_types=[
            pltpu.VMEM_SHARED((_PS, _CW), jnp.float32),
            pltpu.VMEM_SHARED((_PR, _CW), jnp.float32),
            pltpu.VMEM((26, 128), jnp.int32),
            pltpu.VMEM((128, _CW), jnp.float32),
        ],
    )(*ps, *idxs, h, ra, zeros)


def _pad2d(idx, nb, sink):
    return jnp.pad(idx, (0, (nb + 1) * 128 - idx.shape[0]),
                   constant_values=sink).reshape(nb + 1, 128)


def _mm_bias_kernel(x_ref, wT_ref, b_ref, o_ref):
    o_ref[...] = (
        jnp.dot(x_ref[...], wT_ref[...], preferred_element_type=jnp.float32, precision=jax.lax.Precision.HIGHEST)
        + b_ref[0, :]
    )


def _mm_bias(x, wT, b, block=_BLK):
    n, kdim = x.shape
    odim = wT.shape[1]
    return pl.pallas_call(
        _mm_bias_kernel,
        grid=(n // block,),
        in_specs=[
            pl.BlockSpec((block, kdim), lambda i: (i, 0)),
            pl.BlockSpec((kdim, odim), lambda i: (0, 0)),
            pl.BlockSpec((1, odim), lambda i: (0, 0)),
        ],
        out_specs=pl.BlockSpec((block, odim), lambda i: (i, 0)),
        out_shape=jax.ShapeDtypeStruct((n, odim), jnp.float32),
    )(x, wT, b.reshape(1, odim))


def _split_bf16(w):
    hi = w.astype(jnp.bfloat16)
    lo = (w - hi.astype(jnp.float32)).astype(jnp.bfloat16)
    return hi, lo


def _dot3(a, w_hi, w_lo):
    a_hi = a.astype(jnp.bfloat16)
    a_lo = (a - a_hi.astype(jnp.float32)).astype(jnp.bfloat16)
    return (jnp.dot(a_hi, w_hi, preferred_element_type=jnp.float32)
            + (jnp.dot(a_hi, w_lo, preferred_element_type=jnp.float32)
               + jnp.dot(a_lo, w_hi, preferred_element_type=jnp.float32)))


def _gru_kernel(k, *refs):
    m_refs = refs[0:k]
    (wih_h, wih_l, bih_ref, whh_h, whh_l, bhh_ref, w1k_h, w1k_l) = \
        refs[k:k + 8]
    o_refs = refs[k + 8:]
    bhh = bhh_ref[0, :]
    h = None
    for t in range(k):
        gi = _dot3(m_refs[t][...], wih_h[...], wih_l[...]) + bih_ref[0, :]
        if t == 0:
            ir, iz, inn = gi[:, :U], gi[:, U:2 * U], gi[:, 2 * U:]
            r = jax.nn.sigmoid(ir + bhh[:U])
            z = jax.nn.sigmoid(iz + bhh[U:2 * U])
            n = jnp.tanh(inn + r * bhh[2 * U:])
            h = (1.0 - z) * n
        else:
            gh = _dot3(h, whh_h[...], whh_l[...]) + bhh
            r = jax.nn.sigmoid(gi[:, :U] + gh[:, :U])
            z = jax.nn.sigmoid(gi[:, U:2 * U] + gh[:, U:2 * U])
            n = jnp.tanh(gi[:, 2 * U:] + r * gh[:, 2 * U:])
            h = (1.0 - z) * n + z * h
        o_refs[t][...] = _dot3(h, w1k_h[...], w1k_l[...])


def _gru_project(k, msgs_ts, wihT, bih, whhT, bhh, w1kT, block=_BLK):
    nk = 25000
    wih_h, wih_l = _split_bf16(wihT)
    whh_h, whh_l = _split_bf16(whhT)
    w1k_h, w1k_l = _split_bf16(w1kT)
    wspec = pl.BlockSpec((U, 3 * U), lambda i: (0, 0))
    bspec = pl.BlockSpec((1, 3 * U), lambda i: (0, 0))
    pspec = pl.BlockSpec((U, U), lambda i: (0, 0))
    mspec = pl.BlockSpec((block, U), lambda i: (i, 0))
    return pl.pallas_call(
        functools.partial(_gru_kernel, k),
        grid=(nk // block,),
        in_specs=[mspec] * k + [
            wspec, wspec, bspec, wspec, wspec, bspec, pspec, pspec,
        ],
        out_specs=[mspec] * k,
        out_shape=[jax.ShapeDtypeStruct((_NBP * 128, U), jnp.float32)] * k,
    )(*msgs_ts, wih_h, wih_l, bih.reshape(1, 3 * U), whh_h, whh_l,
      bhh.reshape(1, 3 * U), w1k_h, w1k_l)


def _final_kernel(d_ref, g_ref, s_ref, w2T_ref, b2_ref, o_ref):
    pre = d_ref[...] + g_ref[...] + s_ref[...]
    o_ref[...] = (
        jnp.dot(jnp.tanh(pre), w2T_ref[...],
                preferred_element_type=jnp.float32,
                precision=jax.lax.Precision.HIGHEST)
        + b2_ref[0, :]
    )


def _final(d, g, s_pad, w2T, b2, block=_BLK):
    n = d.shape[0]
    return pl.pallas_call(
        _final_kernel,
        grid=(n // block,),
        in_specs=[
            pl.BlockSpec((block, U), lambda i: (i, 0)),
            pl.BlockSpec((block, U), lambda i: (i, 0)),
            pl.BlockSpec((block, U), lambda i: (i, 0)),
            pl.BlockSpec((U, U), lambda i: (0, 0)),
            pl.BlockSpec((1, U), lambda i: (0, 0)),
        ],
        out_specs=pl.BlockSpec((block, U), lambda i: (i, 0)),
        out_shape=jax.ShapeDtypeStruct((n, U), jnp.float32),
    )(d, g, s_pad, w2T, b2.reshape(1, U))


def kernel(h, mem2, mem3, mem4, ring_assign, Wih, Whh, bih, bhh, W1, b1, W2, b2):
    wihT = Wih.T
    whhT = Whh.T
    w1T = W1.T
    w2T = W2.T

    pks = []
    idxs = []
    for ki, (k, mem) in enumerate(((2, mem2), (3, mem3), (4, mem4))):
        msgs_ts = _sc_gather(
            k, h, [_pad2d(mem[:, t], _NBP, 0) for t in range(k)])
        pk_ts = _gru_project(k, msgs_ts, wihT, bih, whhT, bhh,
                             w1T[(2 + ki) * U:(3 + ki) * U, :])
        pks.extend(pk_ts)
        idxs.extend(_pad2d(mem[:, t], _NBP, _PS - 48) for t in range(k))

    zeros = jnp.zeros((_PS, _CW), jnp.float32)
    s_pad, r_pad = _sc_scatter(
        pks, idxs, h, _pad2d(ring_assign, _NBR + 1, NR), zeros)

    pring = _mm_bias(r_pad, w1T[U:2 * U, :], jnp.zeros((U,), jnp.float32),
                     block=640)
    d1 = _mm_bias(h, w1T[:U, :], b1)
    g = pring[ring_assign]
    return _final(d1, g, s_pad, w2T, b2)


# speedup vs baseline: 3.0251x; 1.1683x over previous
import functools

import jax
import jax.numpy as jnp
from jax import lax
from jax.experimental import pallas as pl
from jax.experimental.pallas import tpu as pltpu
from jax.experimental.pallas import tpu_sc as plsc

U = 128
NR = 5000
_BLK = 1000

_PS = 50048
_PR = 5120
_CW = 32


_NBP = 196
_NBR = 390


def _sc_gather_body(k, *refs):
    h = refs[0]
    idxs = refs[1:1 + k]
    outs = refs[1 + k:1 + 2 * k]
    idx_v, data_v, sem = refs[1 + 2 * k:]
    w = lax.axis_index("s") * 2 + lax.axis_index("c")
    per, rem = _NBP // 32, _NBP % 32
    start_w = w * per + jnp.minimum(w, rem)
    nb_w = per + (w < rem).astype(jnp.int32)
    for idx2d, out in zip(idxs, outs):
        pltpu.sync_copy(idx2d.at[pl.ds(start_w, per + 1)],
                        idx_v.at[pl.ds(0, per + 1)])

        def body(i, carry, out=out):
            b = start_w + i
            pltpu.async_copy(h.at[idx_v.at[i]], data_v, sem).wait()
            pltpu.sync_copy(data_v, out.at[pl.ds(b * 128, 128)])
            return carry

        lax.fori_loop(0, nb_w, body, 0)


def _sc_gather(k, h, idxs):
    return pl.kernel(
        functools.partial(_sc_gather_body, k),
        out_type=tuple(jax.ShapeDtypeStruct((_NBP * 128, 128), jnp.float32)
                       for _ in range(k)),
        mesh=plsc.VectorSubcoreMesh(core_axis_name="c", subcore_axis_name="s"),
        compiler_params=pltpu.CompilerParams(use_tc_tiling_on_sc=False),
        scratch_types=[
            pltpu.VMEM((7, 128), jnp.int32),
            pltpu.VMEM((128, 128), jnp.float32),
            pltpu.SemaphoreType.DMA,
        ],
    )(h, *idxs)


def _sc_ring_body(hsrc, ridx, zeros_hbm, r_out, racc, idx_v, data_v):
    c = lax.axis_index("c")
    s = lax.axis_index("s")
    for cc in range(2):
        col0 = (2 * c + cc) * _CW

        @pl.when(s == 0)
        def _():
            pltpu.sync_copy(zeros_hbm.at[pl.ds(0, _PR)], racc)

        plsc.subcore_barrier()
        perr, remr = _NBR // 16, _NBR % 16
        start_r = s * perr + jnp.minimum(s, remr)
        nbr_w = perr + (s < remr).astype(jnp.int32)
        pltpu.sync_copy(ridx.at[pl.ds(start_r, perr + 1)],
                        idx_v.at[pl.ds(0, perr + 1)])

        def rbody(i, carry, col0=col0, start_r=start_r):
            b = start_r + i
            pltpu.sync_copy(
                hsrc.at[pl.ds(b * 128, 128), pl.ds(col0, _CW)], data_v)
            pltpu.sync_copy(data_v, racc.at[idx_v.at[i]], add=True)
            return carry

        lax.fori_loop(0, nbr_w, rbody, 0)

        @pl.when(s == 15)
        def _(col0=col0):
            pltpu.sync_copy(ridx.at[pl.ds(_NBR, 1)], idx_v.at[pl.ds(25, 1)])
            pltpu.sync_copy(
                hsrc.at[pl.ds(_NBR * 128, 80), pl.ds(col0, _CW)],
                data_v.at[pl.ds(0, 80)])
            pltpu.sync_copy(data_v, racc.at[idx_v.at[25]], add=True)

        plsc.subcore_barrier()

        @pl.when(s == 0)
        def _():
            pltpu.sync_copy(racc, r_out.at[:, pl.ds(col0, _CW)])

        plsc.subcore_barrier()


def _sc_ring(h, ra, zeros):
    return pl.kernel(
        _sc_ring_body,
        out_type=jax.ShapeDtypeStruct((_PR, 128), jnp.float32),
        mesh=plsc.VectorSubcoreMesh(core_axis_name="c", subcore_axis_name="s"),
        compiler_params=pltpu.CompilerParams(use_tc_tiling_on_sc=False),
        scratch_types=[
            pltpu.VMEM_SHARED((_PR, _CW), jnp.float32),
            pltpu.VMEM((26, 128), jnp.int32),
            pltpu.VMEM((128, _CW), jnp.float32),
        ],
    )(h, ra, zeros)


def _sc_scatter_body(*refs):
    ps = refs[0:9]
    idxs = refs[9:18]
    zeros_hbm, s_out = refs[18:20]
    sacc, idx_v, data_v = refs[20:23]
    c = lax.axis_index("c")
    s = lax.axis_index("s")
    for cc in range(2):
        col0 = (2 * c + cc) * _CW

        @pl.when(s == 0)
        def _():
            pltpu.sync_copy(zeros_hbm, sacc)

        plsc.subcore_barrier()
        per, rem = _NBP // 16, _NBP % 16
        start_w = s * per + jnp.minimum(s, rem)
        nb_w = per + (s < rem).astype(jnp.int32)
        for src, idx2d in zip(ps, idxs):

            def body(i, carry, src=src, col0=col0, start_w=start_w):
                b = start_w + i
                pltpu.sync_copy(
                    src.at[pl.ds(b * 128, 128), pl.ds(col0, _CW)], data_v)
                pltpu.sync_copy(data_v, sacc.at[idx_v.at[i]], add=True)
                return carry

            pltpu.sync_copy(idx2d.at[pl.ds(start_w, per + 1)],
                            idx_v.at[pl.ds(0, per + 1)])
            lax.fori_loop(0, nb_w, body, 0)

        plsc.subcore_barrier()

        @pl.when(s == 0)
        def _():
            pltpu.sync_copy(sacc, s_out.at[:, pl.ds(col0, _CW)])

        plsc.subcore_barrier()


def _sc_scatter(ps, idxs, zeros):
    return pl.kernel(
        _sc_scatter_body,
        out_type=jax.ShapeDtypeStruct((_PS, 128), jnp.float32),
        mesh=plsc.VectorSubcoreMesh(core_axis_name="c", subcore_axis_name="s"),
        compiler_params=pltpu.CompilerParams(use_tc_tiling_on_sc=False),
        scratch_types=[
            pltpu.VMEM_SHARED((_PS, _CW), jnp.float32),
            pltpu.VMEM((13, 128), jnp.int32),
            pltpu.VMEM((128, _CW), jnp.float32),
        ],
    )(*ps, *idxs, zeros)


def _pad2d(idx, nb, sink):
    return jnp.pad(idx, (0, (nb + 1) * 128 - idx.shape[0]),
                   constant_values=sink).reshape(nb + 1, 128)


def _mm_bias_kernel(x_ref, wT_ref, b_ref, o_ref):
    o_ref[...] = (
        jnp.dot(x_ref[...], wT_ref[...], preferred_element_type=jnp.float32, precision=jax.lax.Precision.HIGHEST)
        + b_ref[0, :]
    )


def _mm_bias(x, wT, b, block=_BLK):
    n, kdim = x.shape
    odim = wT.shape[1]
    return pl.pallas_call(
        _mm_bias_kernel,
        grid=(n // block,),
        in_specs=[
            pl.BlockSpec((block, kdim), lambda i: (i, 0)),
            pl.BlockSpec((kdim, odim), lambda i: (0, 0)),
            pl.BlockSpec((1, odim), lambda i: (0, 0)),
        ],
        out_specs=pl.BlockSpec((block, odim), lambda i: (i, 0)),
        out_shape=jax.ShapeDtypeStruct((n, odim), jnp.float32),
    )(x, wT, b.reshape(1, odim))


def _split_bf16(w):
    hi = w.astype(jnp.bfloat16)
    lo = (w - hi.astype(jnp.float32)).astype(jnp.bfloat16)
    return hi, lo


def _dot3(a, w_hi, w_lo):
    a_hi = a.astype(jnp.bfloat16)
    a_lo = (a - a_hi.astype(jnp.float32)).astype(jnp.bfloat16)
    return (jnp.dot(a_hi, w_hi, preferred_element_type=jnp.float32)
            + (jnp.dot(a_hi, w_lo, preferred_element_type=jnp.float32)
               + jnp.dot(a_lo, w_hi, preferred_element_type=jnp.float32)))


def _gru_kernel(k, *refs):
    m_refs = refs[0:k]
    (wih_h, wih_l, bih_ref, whh_h, whh_l, bhh_ref, w1k_h, w1k_l) = \
        refs[k:k + 8]
    o_refs = refs[k + 8:]
    bhh = bhh_ref[0, :]
    h = None
    for t in range(k):
        gi = _dot3(m_refs[t][...], wih_h[...], wih_l[...]) + bih_ref[0, :]
        if t == 0:
            ir, iz, inn = gi[:, :U], gi[:, U:2 * U], gi[:, 2 * U:]
            r = jax.nn.sigmoid(ir + bhh[:U])
            z = jax.nn.sigmoid(iz + bhh[U:2 * U])
            n = jnp.tanh(inn + r * bhh[2 * U:])
            h = (1.0 - z) * n
        else:
            gh = _dot3(h, whh_h[...], whh_l[...]) + bhh
            r = jax.nn.sigmoid(gi[:, :U] + gh[:, :U])
            z = jax.nn.sigmoid(gi[:, U:2 * U] + gh[:, U:2 * U])
            n = jnp.tanh(gi[:, 2 * U:] + r * gh[:, 2 * U:])
            h = (1.0 - z) * n + z * h
        o_refs[t][...] = _dot3(h, w1k_h[...], w1k_l[...])


def _gru_project(k, msgs_ts, wihT, bih, whhT, bhh, w1kT, block=_BLK):
    nk = 25000
    wih_h, wih_l = _split_bf16(wihT)
    whh_h, whh_l = _split_bf16(whhT)
    w1k_h, w1k_l = _split_bf16(w1kT)
    wspec = pl.BlockSpec((U, 3 * U), lambda i: (0, 0))
    bspec = pl.BlockSpec((1, 3 * U), lambda i: (0, 0))
    pspec = pl.BlockSpec((U, U), lambda i: (0, 0))
    mspec = pl.BlockSpec((block, U), lambda i: (i, 0))
    return pl.pallas_call(
        functools.partial(_gru_kernel, k),
        grid=(nk // block,),
        in_specs=[mspec] * k + [
            wspec, wspec, bspec, wspec, wspec, bspec, pspec, pspec,
        ],
        out_specs=[mspec] * k,
        out_shape=[jax.ShapeDtypeStruct((_NBP * 128, U), jnp.float32)] * k,
    )(*msgs_ts, wih_h, wih_l, bih.reshape(1, 3 * U), whh_h, whh_l,
      bhh.reshape(1, 3 * U), w1k_h, w1k_l)


def _final_kernel(d_ref, g_ref, s_ref, w2T_ref, b2_ref, o_ref):
    pre = d_ref[...] + g_ref[...] + s_ref[...]
    o_ref[...] = (
        jnp.dot(jnp.tanh(pre), w2T_ref[...],
                preferred_element_type=jnp.float32,
                precision=jax.lax.Precision.HIGHEST)
        + b2_ref[0, :]
    )


def _final(d, g, s_pad, w2T, b2, block=_BLK):
    n = d.shape[0]
    return pl.pallas_call(
        _final_kernel,
        grid=(n // block,),
        in_specs=[
            pl.BlockSpec((block, U), lambda i: (i, 0)),
            pl.BlockSpec((block, U), lambda i: (i, 0)),
            pl.BlockSpec((block, U), lambda i: (i, 0)),
            pl.BlockSpec((U, U), lambda i: (0, 0)),
            pl.BlockSpec((1, U), lambda i: (0, 0)),
        ],
        out_specs=pl.BlockSpec((block, U), lambda i: (i, 0)),
        out_shape=jax.ShapeDtypeStruct((n, U), jnp.float32),
    )(d, g, s_pad, w2T, b2.reshape(1, U))


def kernel(h, mem2, mem3, mem4, ring_assign, Wih, Whh, bih, bhh, W1, b1, W2, b2):
    wihT = Wih.T
    whhT = Whh.T
    w1T = W1.T
    w2T = W2.T

    zeros = jnp.zeros((_PS, _CW), jnp.float32)
    pks = []
    idxs = []
    for ki, (k, mem) in enumerate(((2, mem2), (3, mem3), (4, mem4))):
        msgs_ts = _sc_gather(
            k, h, [_pad2d(mem[:, t], _NBP, 0) for t in range(k)])
        pk_ts = _gru_project(k, msgs_ts, wihT, bih, whhT, bhh,
                             w1T[(2 + ki) * U:(3 + ki) * U, :])
        pks.extend(pk_ts)
        idxs.extend(_pad2d(mem[:, t], _NBP, _PS - 48) for t in range(k))

    r_pad = _sc_ring(h, _pad2d(ring_assign, _NBR + 1, NR), zeros)
    s_pad = _sc_scatter(pks, idxs, zeros)

    pring = _mm_bias(r_pad, w1T[U:2 * U, :], jnp.zeros((U,), jnp.float32),
                     block=640)
    d1 = _mm_bias(h, w1T[:U, :], b1)
    g = pring[ring_assign]
    return _final(d1, g, s_pad, w2T, b2)


# speedup vs baseline: 3.1693x; 1.0476x over previous
import functools

import jax
import jax.numpy as jnp
from jax import lax
from jax.experimental import pallas as pl
from jax.experimental.pallas import tpu as pltpu
from jax.experimental.pallas import tpu_sc as plsc

U = 128
NR = 5000
_BLK = 1000

_PS = 50048
_PR = 5120
_CW = 32


_NBP = 196
_NBR = 390


def _sc_gather_body(k, *refs):
    h = refs[0]
    idxs = refs[1:1 + k]
    outs = refs[1 + k:1 + 2 * k]
    idx_v, data_v, sem = refs[1 + 2 * k:]
    w = lax.axis_index("s") * 2 + lax.axis_index("c")
    per, rem = _NBP // 32, _NBP % 32
    start_w = w * per + jnp.minimum(w, rem)
    nb_w = per + (w < rem).astype(jnp.int32)
    for idx2d, out in zip(idxs, outs):
        pltpu.sync_copy(idx2d.at[pl.ds(start_w, per + 1)],
                        idx_v.at[pl.ds(0, per + 1)])

        def body(i, carry, out=out):
            b = start_w + i
            pltpu.async_copy(h.at[idx_v.at[i]], data_v, sem).wait()
            pltpu.sync_copy(data_v, out.at[pl.ds(b * 128, 128)])
            return carry

        lax.fori_loop(0, nb_w, body, 0)


def _sc_gather(k, h, idxs):
    return pl.kernel(
        functools.partial(_sc_gather_body, k),
        out_type=tuple(jax.ShapeDtypeStruct((_NBP * 128, 128), jnp.float32)
                       for _ in range(k)),
        mesh=plsc.VectorSubcoreMesh(core_axis_name="c", subcore_axis_name="s"),
        compiler_params=pltpu.CompilerParams(use_tc_tiling_on_sc=False),
        scratch_types=[
            pltpu.VMEM((7, 128), jnp.int32),
            pltpu.VMEM((128, 128), jnp.float32),
            pltpu.SemaphoreType.DMA,
        ],
    )(h, *idxs)


def _sc_ring_body(hsrc, ridx, zeros_hbm, r_out, racc, idx_v, data_v):
    c = lax.axis_index("c")
    s = lax.axis_index("s")
    for cc in range(2):
        col0 = (2 * c + cc) * _CW

        @pl.when(s == 0)
        def _():
            pltpu.sync_copy(zeros_hbm.at[pl.ds(0, _PR)], racc)

        plsc.subcore_barrier()
        perr, remr = _NBR // 16, _NBR % 16
        start_r = s * perr + jnp.minimum(s, remr)
        nbr_w = perr + (s < remr).astype(jnp.int32)
        pltpu.sync_copy(ridx.at[pl.ds(start_r, perr + 1)],
                        idx_v.at[pl.ds(0, perr + 1)])

        def rbody(i, carry, col0=col0, start_r=start_r):
            b = start_r + i
            pltpu.sync_copy(
                hsrc.at[pl.ds(b * 128, 128), pl.ds(col0, _CW)], data_v)
            pltpu.sync_copy(data_v, racc.at[idx_v.at[i]], add=True)
            return carry

        lax.fori_loop(0, nbr_w, rbody, 0)

        @pl.when(s == 15)
        def _(col0=col0):
            pltpu.sync_copy(ridx.at[pl.ds(_NBR, 1)], idx_v.at[pl.ds(25, 1)])
            pltpu.sync_copy(
                hsrc.at[pl.ds(_NBR * 128, 80), pl.ds(col0, _CW)],
                data_v.at[pl.ds(0, 80)])
            pltpu.sync_copy(data_v, racc.at[idx_v.at[25]], add=True)

        plsc.subcore_barrier()

        @pl.when(s == 0)
        def _():
            pltpu.sync_copy(racc, r_out.at[:, pl.ds(col0, _CW)])

        plsc.subcore_barrier()


def _sc_ring(h, ra, zeros):
    return pl.kernel(
        _sc_ring_body,
        out_type=jax.ShapeDtypeStruct((_PR, 128), jnp.float32),
        mesh=plsc.VectorSubcoreMesh(core_axis_name="c", subcore_axis_name="s"),
        compiler_params=pltpu.CompilerParams(use_tc_tiling_on_sc=False),
        scratch_types=[
            pltpu.VMEM_SHARED((_PR, _CW), jnp.float32),
            pltpu.VMEM((26, 128), jnp.int32),
            pltpu.VMEM((128, _CW), jnp.float32),
        ],
    )(h, ra, zeros)


def _sc_scatter_body(*refs):
    ps = refs[0:9]
    idxs = refs[9:18]
    zeros_hbm, s_out = refs[18:20]
    sacc, idx_v, data_v, fsem = refs[20:24]
    c = lax.axis_index("c")
    s = lax.axis_index("s")
    for cc in range(2):
        col0 = (2 * c + cc) * _CW

        @pl.when(s == 0)
        def _():
            pltpu.sync_copy(zeros_hbm, sacc)

        plsc.subcore_barrier()
        per, rem = _NBP // 16, _NBP % 16
        start_w = s * per + jnp.minimum(s, rem)
        nb_w = per + (s < rem).astype(jnp.int32)
        for src, idx2d in zip(ps, idxs):

            def fetch(i, slot, src=src, col0=col0, start_w=start_w):
                pltpu.async_copy(
                    src.at[pl.ds((start_w + i) * 128, 128),
                           pl.ds(col0, _CW)],
                    data_v.at[slot], fsem.at[slot])

            def body(i, carry, src=src, col0=col0, start_w=start_w,
                     fetch=fetch):
                slot = lax.rem(i, 2)
                pltpu.make_async_copy(
                    src.at[pl.ds(start_w * 128, 128), pl.ds(col0, _CW)],
                    data_v.at[slot], fsem.at[slot]).wait()

                @pl.when(i + 1 < nb_w)
                def _():
                    fetch(i + 1, 1 - slot)

                pltpu.sync_copy(data_v.at[slot], sacc.at[idx_v.at[i]],
                                add=True)
                return carry

            pltpu.sync_copy(idx2d.at[pl.ds(start_w, per + 1)],
                            idx_v.at[pl.ds(0, per + 1)])

            @pl.when(nb_w > 0)
            def _(fetch=fetch):
                fetch(0, 0)

            lax.fori_loop(0, nb_w, body, 0)

        plsc.subcore_barrier()

        @pl.when(s == 0)
        def _():
            pltpu.sync_copy(sacc, s_out.at[:, pl.ds(col0, _CW)])

        plsc.subcore_barrier()


def _sc_scatter(ps, idxs, zeros):
    return pl.kernel(
        _sc_scatter_body,
        out_type=jax.ShapeDtypeStruct((_PS, 128), jnp.float32),
        mesh=plsc.VectorSubcoreMesh(core_axis_name="c", subcore_axis_name="s"),
        compiler_params=pltpu.CompilerParams(use_tc_tiling_on_sc=False),
        scratch_types=[
            pltpu.VMEM_SHARED((_PS, _CW), jnp.float32),
            pltpu.VMEM((13, 128), jnp.int32),
            pltpu.VMEM((2, 128, _CW), jnp.float32),
            pltpu.SemaphoreType.DMA((2,)),
        ],
    )(*ps, *idxs, zeros)


def _pad2d(idx, nb, sink):
    return jnp.pad(idx, (0, (nb + 1) * 128 - idx.shape[0]),
                   constant_values=sink).reshape(nb + 1, 128)


def _mm_bias_kernel(x_ref, wT_ref, b_ref, o_ref):
    o_ref[...] = (
        jnp.dot(x_ref[...], wT_ref[...], preferred_element_type=jnp.float32, precision=jax.lax.Precision.HIGHEST)
        + b_ref[0, :]
    )


def _mm_bias(x, wT, b, block=_BLK):
    n, kdim = x.shape
    odim = wT.shape[1]
    return pl.pallas_call(
        _mm_bias_kernel,
        grid=(n // block,),
        in_specs=[
            pl.BlockSpec((block, kdim), lambda i: (i, 0)),
            pl.BlockSpec((kdim, odim), lambda i: (0, 0)),
            pl.BlockSpec((1, odim), lambda i: (0, 0)),
        ],
        out_specs=pl.BlockSpec((block, odim), lambda i: (i, 0)),
        out_shape=jax.ShapeDtypeStruct((n, odim), jnp.float32),
    )(x, wT, b.reshape(1, odim))


def _split_bf16(w):
    hi = w.astype(jnp.bfloat16)
    lo = (w - hi.astype(jnp.float32)).astype(jnp.bfloat16)
    return hi, lo


def _dot3(a, w_hi, w_lo):
    a_hi = a.astype(jnp.bfloat16)
    a_lo = (a - a_hi.astype(jnp.float32)).astype(jnp.bfloat16)
    return (jnp.dot(a_hi, w_hi, preferred_element_type=jnp.float32)
            + (jnp.dot(a_hi, w_lo, preferred_element_type=jnp.float32)
               + jnp.dot(a_lo, w_hi, preferred_element_type=jnp.float32)))


def _gru_kernel(k, *refs):
    m_refs = refs[0:k]
    (wih_h, wih_l, bih_ref, whh_h, whh_l, bhh_ref, w1k_h, w1k_l) = \
        refs[k:k + 8]
    o_refs = refs[k + 8:]
    bhh = bhh_ref[0, :]
    h = None
    for t in range(k):
        gi = _dot3(m_refs[t][...], wih_h[...], wih_l[...]) + bih_ref[0, :]
        if t == 0:
            ir, iz, inn = gi[:, :U], gi[:, U:2 * U], gi[:, 2 * U:]
            r = jax.nn.sigmoid(ir + bhh[:U])
            z = jax.nn.sigmoid(iz + bhh[U:2 * U])
            n = jnp.tanh(inn + r * bhh[2 * U:])
            h = (1.0 - z) * n
        else:
            gh = _dot3(h, whh_h[...], whh_l[...]) + bhh
            r = jax.nn.sigmoid(gi[:, :U] + gh[:, :U])
            z = jax.nn.sigmoid(gi[:, U:2 * U] + gh[:, U:2 * U])
            n = jnp.tanh(gi[:, 2 * U:] + r * gh[:, 2 * U:])
            h = (1.0 - z) * n + z * h
        o_refs[t][...] = _dot3(h, w1k_h[...], w1k_l[...])


def _gru_project(k, msgs_ts, wihT, bih, whhT, bhh, w1kT, block=_BLK):
    nk = 25000
    wih_h, wih_l = _split_bf16(wihT)
    whh_h, whh_l = _split_bf16(whhT)
    w1k_h, w1k_l = _split_bf16(w1kT)
    wspec = pl.BlockSpec((U, 3 * U), lambda i: (0, 0))
    bspec = pl.BlockSpec((1, 3 * U), lambda i: (0, 0))
    pspec = pl.BlockSpec((U, U), lambda i: (0, 0))
    mspec = pl.BlockSpec((block, U), lambda i: (i, 0))
    return pl.pallas_call(
        functools.partial(_gru_kernel, k),
        grid=(nk // block,),
        in_specs=[mspec] * k + [
            wspec, wspec, bspec, wspec, wspec, bspec, pspec, pspec,
        ],
        out_specs=[mspec] * k,
        out_shape=[jax.ShapeDtypeStruct((_NBP * 128, U), jnp.float32)] * k,
    )(*msgs_ts, wih_h, wih_l, bih.reshape(1, 3 * U), whh_h, whh_l,
      bhh.reshape(1, 3 * U), w1k_h, w1k_l)


def _final_kernel(d_ref, g_ref, s_ref, w2T_ref, b2_ref, o_ref):
    pre = d_ref[...] + g_ref[...] + s_ref[...]
    o_ref[...] = (
        jnp.dot(jnp.tanh(pre), w2T_ref[...],
                preferred_element_type=jnp.float32,
                precision=jax.lax.Precision.HIGHEST)
        + b2_ref[0, :]
    )


def _final(d, g, s_pad, w2T, b2, block=_BLK):
    n = d.shape[0]
    return pl.pallas_call(
        _final_kernel,
        grid=(n // block,),
        in_specs=[
            pl.BlockSpec((block, U), lambda i: (i, 0)),
            pl.BlockSpec((block, U), lambda i: (i, 0)),
            pl.BlockSpec((block, U), lambda i: (i, 0)),
            pl.BlockSpec((U, U), lambda i: (0, 0)),
            pl.BlockSpec((1, U), lambda i: (0, 0)),
        ],
        out_specs=pl.BlockSpec((block, U), lambda i: (i, 0)),
        out_shape=jax.ShapeDtypeStruct((n, U), jnp.float32),
    )(d, g, s_pad, w2T, b2.reshape(1, U))


def kernel(h, mem2, mem3, mem4, ring_assign, Wih, Whh, bih, bhh, W1, b1, W2, b2):
    wihT = Wih.T
    whhT = Whh.T
    w1T = W1.T
    w2T = W2.T

    zeros = jnp.zeros((_PS, _CW), jnp.float32)
    pks = []
    idxs = []
    for ki, (k, mem) in enumerate(((2, mem2), (3, mem3), (4, mem4))):
        msgs_ts = _sc_gather(
            k, h, [_pad2d(mem[:, t], _NBP, 0) for t in range(k)])
        pk_ts = _gru_project(k, msgs_ts, wihT, bih, whhT, bhh,
                             w1T[(2 + ki) * U:(3 + ki) * U, :])
        pks.extend(pk_ts)
        idxs.extend(_pad2d(mem[:, t], _NBP, _PS - 48) for t in range(k))

    r_pad = _sc_ring(h, _pad2d(ring_assign, _NBR + 1, NR), zeros)
    s_pad = _sc_scatter(pks, idxs, zeros)

    pring = _mm_bias(r_pad, w1T[U:2 * U, :], jnp.zeros((U,), jnp.float32),
                     block=640)
    d1 = _mm_bias(h, w1T[:U, :], b1)
    g = pring[ring_assign]
    return _final(d1, g, s_pad, w2T, b2)
